# Initial kernel scaffold; baseline (speedup 1.0000x reference)
import functools

import jax
import jax.numpy as jnp
from jax.experimental import pallas as pl
from jax.experimental.pallas import tpu as pltpu
from jax.experimental.pallas import tpu_sc as plsc

T, H, I, E, K = 2048, 1024, 4096, 8, 2
SOFTCAP = 30.0
BM = 128
C = T * K + E * BM
NT = C // BM
NJ = 2
IH = I // NJ
DWIN = 32
CWIN = 8


def _routing_body(x_ref, g_ref, dest_ref, w_ref, pc_ref):
    x = x_ref[...]
    gw = g_ref[...]
    x_hi = x.astype(jnp.bfloat16)
    x_lo = (x - x_hi.astype(jnp.float32)).astype(jnp.bfloat16)
    g_hi = gw.astype(jnp.bfloat16)
    g_lo = (gw - g_hi.astype(jnp.float32)).astype(jnp.bfloat16)
    logits = (
        jnp.dot(x_hi, g_hi, preferred_element_type=jnp.float32)
        + jnp.dot(x_hi, g_lo, preferred_element_type=jnp.float32)
        + jnp.dot(x_lo, g_hi, preferred_element_type=jnp.float32)
    )
    logits = jnp.tanh(logits * (1.0 / SOFTCAP)) * SOFTCAP

    m = jnp.max(logits, axis=-1, keepdims=True)
    ex = jnp.exp(logits - m)
    p = ex / jnp.sum(ex, axis=-1, keepdims=True)

    lane = jax.lax.broadcasted_iota(jnp.int32, (T, E), 1)
    big = jnp.int32(E)
    c0 = jnp.where(logits == m, lane, big)
    i0 = jnp.min(c0, axis=-1, keepdims=True)
    sel0 = lane == i0
    l1 = jnp.where(sel0, -jnp.inf, logits)
    m1 = jnp.max(l1, axis=-1, keepdims=True)
    c1 = jnp.where(l1 == m1, lane, big)
    i1 = jnp.min(c1, axis=-1, keepdims=True)
    sel1 = lane == i1

    w0 = jnp.sum(jnp.where(sel0, p, 0.0), axis=-1, keepdims=True)
    w1 = jnp.sum(jnp.where(sel1, p, 0.0), axis=-1, keepdims=True)

    mask = (sel0 | sel1).astype(jnp.bfloat16)
    ti = jax.lax.broadcasted_iota(jnp.int32, (T, T), 0)
    tj = jax.lax.broadcasted_iota(jnp.int32, (T, T), 1)
    tril = (tj < ti).astype(jnp.bfloat16)
    rank = jnp.dot(tril, mask, preferred_element_type=jnp.float32)

    counts = jnp.sum(mask.astype(jnp.float32), axis=0, keepdims=True)
    padded = jnp.ceil(counts * (1.0 / BM)) * BM
    p8 = jnp.broadcast_to(padded, (E, E)).astype(jnp.bfloat16)
    ei = jax.lax.broadcasted_iota(jnp.int32, (E, E), 0)
    ej = jax.lax.broadcasted_iota(jnp.int32, (E, E), 1)
    tru = (ei < ej).astype(jnp.bfloat16)
    base8 = jnp.dot(p8, tru, preferred_element_type=jnp.float32)
    base = base8[0:1, :]

    posv = base + rank
    d0 = jnp.sum(jnp.where(sel0, posv, 0.0), axis=-1, keepdims=True)
    d1 = jnp.sum(jnp.where(sel1, posv, 0.0), axis=-1, keepdims=True)

    dest_ref[:, 0:1] = d0.astype(jnp.int32)
    dest_ref[:, 1:2] = d1.astype(jnp.int32)
    w_ref[:, 0:1] = w0
    w_ref[:, 1:2] = w1
    pc_ref[...] = padded.astype(jnp.int32)


def _routing(x, gate_w):
    return pl.pallas_call(
        _routing_body,
        out_shape=(
            jax.ShapeDtypeStruct((T, K), jnp.int32),
            jax.ShapeDtypeStruct((T, K), jnp.float32),
            jax.ShapeDtypeStruct((1, E), jnp.int32),
        ),
    )(x, gate_w)


def _dispatch(x, dest_flat):
    mesh = plsc.VectorSubcoreMesh(core_axis_name="core", subcore_axis_name="subcore")

    @functools.partial(
        pl.kernel,
        out_type=jax.ShapeDtypeStruct((C, H), jnp.float32),
        mesh=mesh,
    )
    def k(x_hbm, d_hbm, o_hbm):
        def body(x_vmem, i_vmem):
            pltpu.sync_copy(x_vmem, o_hbm.at[i_vmem.at[0]])

        pltpu.emit_pipeline(
            body,
            grid=(K * T // DWIN,),
            in_specs=[
                pl.BlockSpec((DWIN, H), lambda i: (jax.lax.rem(i, T // DWIN), 0)),
                pl.BlockSpec((1, DWIN), lambda i: (0, i)),
            ],
            out_specs=[],
            core_axis_name=("core", "subcore"),
            dimension_semantics=(pltpu.PARALLEL,),
        )(x_hbm, d_hbm)

    return k(x, dest_flat)


def _grouped_body(eot_ref, x_ref, wi0_ref, wi1_ref, wo_ref, y_ref):
    x = x_ref[...].astype(jnp.bfloat16)
    w0 = wi0_ref[0].astype(jnp.bfloat16)
    w1 = wi1_ref[0].astype(jnp.bfloat16)
    g = jnp.dot(x, w0, preferred_element_type=jnp.float32)
    u = jnp.dot(x, w1, preferred_element_type=jnp.float32)
    h = (jax.nn.gelu(g, approximate=True) * u).astype(jnp.bfloat16)
    wo_b = wo_ref[0].astype(jnp.bfloat16)
    y_ref[0] = jnp.dot(h, wo_b, preferred_element_type=jnp.float32)


def _grouped(x_sorted, eot, wi_0, wi_1, wo):
    grid_spec = pltpu.PrefetchScalarGridSpec(
        num_scalar_prefetch=1,
        grid=(NJ, NT),
        in_specs=[
            pl.BlockSpec((BM, H), lambda j, i, eot: (i, 0)),
            pl.BlockSpec((1, H, IH), lambda j, i, eot: (eot[i], 0, j)),
            pl.BlockSpec((1, H, IH), lambda j, i, eot: (eot[i], 0, j)),
            pl.BlockSpec((1, IH, H), lambda j, i, eot: (eot[i], j, 0)),
        ],
        out_specs=pl.BlockSpec((1, BM, H), lambda j, i, eot: (j, i, 0)),
    )
    return pl.pallas_call(
        _grouped_body,
        grid_spec=grid_spec,
        out_shape=jax.ShapeDtypeStruct((NJ, C, H), jnp.float32),
    )(eot, x_sorted, wi_0, wi_1, wo)


def _combine(y2, d0, d0b, d1, d1b, w0b, w1b):
    mesh = plsc.VectorSubcoreMesh(core_axis_name="core", subcore_axis_name="subcore")

    @functools.partial(
        pl.kernel,
        out_type=jax.ShapeDtypeStruct((T, H), jnp.float32),
        mesh=mesh,
        scratch_types=[
            pltpu.VMEM((CWIN, H), jnp.float32),
            pltpu.VMEM((CWIN, H), jnp.float32),
            pltpu.VMEM((CWIN, H), jnp.float32),
            pltpu.VMEM((CWIN, H), jnp.float32),
        ],
    )
    def k(y_hbm, d0_hbm, d0b_hbm, d1_hbm, d1b_hbm, w0_hbm, w1_hbm, o_hbm,
          a0_vmem, a1_vmem, b0_vmem, b1_vmem):
        def body(d0_v, d0b_v, d1_v, d1b_v, w0_v, w1_v, o_v):
            pltpu.sync_copy(y_hbm.at[d0_v.at[0]], a0_vmem)
            pltpu.sync_copy(y_hbm.at[d0b_v.at[0]], a1_vmem)
            pltpu.sync_copy(y_hbm.at[d1_v.at[0]], b0_vmem)
            pltpu.sync_copy(y_hbm.at[d1b_v.at[0]], b1_vmem)

            @pl.loop(0, CWIN)
            def _(r):
                @pl.loop(0, H, step=16)
                def _(c):
                    slc = (pl.ds(r, 1), pl.ds(c, 16))
                    wslc = (pl.ds(r, 1), pl.ds(0, 16))
                    a = a0_vmem.at[*slc][...] + a1_vmem.at[*slc][...]
                    b = b0_vmem.at[*slc][...] + b1_vmem.at[*slc][...]
                    o_v.at[*slc][...] = (a * w0_v.at[*wslc][...]
                                         + b * w1_v.at[*wslc][...])

        pltpu.emit_pipeline(
            body,
            grid=(T // CWIN,),
            in_specs=[
                pl.BlockSpec((1, CWIN), lambda i: (0, i)),
                pl.BlockSpec((1, CWIN), lambda i: (0, i)),
                pl.BlockSpec((1, CWIN), lambda i: (0, i)),
                pl.BlockSpec((1, CWIN), lambda i: (0, i)),
                pl.BlockSpec((CWIN, 16), lambda i: (i, 0)),
                pl.BlockSpec((CWIN, 16), lambda i: (i, 0)),
            ],
            out_specs=[pl.BlockSpec((CWIN, H), lambda i: (i, 0))],
            core_axis_name=("core", "subcore"),
            dimension_semantics=(pltpu.PARALLEL,),
        )(d0_hbm, d0b_hbm, d1_hbm, d1b_hbm, w0_hbm, w1_hbm, o_hbm)

    return k(y2, d0, d0b, d1, d1b, w0b, w1b)


def kernel(hidden_states, gate_w, wi_0, wi_1, wo):
    dest, wts, pc = _routing(hidden_states, gate_w)

    tiles_pe = pc[0] // BM
    cumt = jnp.cumsum(tiles_pe)
    eot = jnp.minimum(
        jnp.searchsorted(cumt, jnp.arange(NT, dtype=jnp.int32), side="right"),
        E - 1,
    ).astype(jnp.int32)

    dest_flat = dest.T.reshape(1, K * T)
    x_sorted = _dispatch(hidden_states, dest_flat)

    y = _grouped(x_sorted, eot, wi_0, wi_1, wo)

    y2 = y.reshape(NJ * C, H)
    d0 = dest[:, 0].reshape(1, T)
    d1 = dest[:, 1].reshape(1, T)
    d0b = d0 + C
    d1b = d1 + C
    w0b = jnp.broadcast_to(wts[:, 0:1], (T, 16))
    w1b = jnp.broadcast_to(wts[:, 1:2], (T, 16))
    return _combine(y2, d0, d0b, d1, d1b, w0b, w1b)


# speedup vs baseline: 1.5712x; 1.5712x over previous
import functools

import jax
import jax.numpy as jnp
from jax.experimental import pallas as pl
from jax.experimental.pallas import tpu as pltpu
from jax.experimental.pallas import tpu_sc as plsc

T, H, I, E, K = 2048, 1024, 4096, 8, 2
SOFTCAP = 30.0
BM = 128
C = T * K + E * BM
NT = C // BM
NJ = 2
IH = I // NJ


def _routing_body(x_ref, g_ref, dest_ref, w_ref, pc_ref):
    x = x_ref[...]
    gw = g_ref[...]
    logits = jnp.dot(x.astype(jnp.bfloat16), gw.astype(jnp.bfloat16),
                     preferred_element_type=jnp.float32)
    logits = jnp.tanh(logits * (1.0 / SOFTCAP)) * SOFTCAP

    m = jnp.max(logits, axis=-1, keepdims=True)
    ex = jnp.exp(logits - m)
    p = ex / jnp.sum(ex, axis=-1, keepdims=True)

    lane = jax.lax.broadcasted_iota(jnp.int32, (T, E), 1)
    big = jnp.int32(E)
    c0 = jnp.where(logits == m, lane, big)
    i0 = jnp.min(c0, axis=-1, keepdims=True)
    sel0 = lane == i0
    l1 = jnp.where(sel0, -jnp.inf, logits)
    m1 = jnp.max(l1, axis=-1, keepdims=True)
    c1 = jnp.where(l1 == m1, lane, big)
    i1 = jnp.min(c1, axis=-1, keepdims=True)
    sel1 = lane == i1

    w0 = jnp.sum(jnp.where(sel0, p, 0.0), axis=-1, keepdims=True)
    w1 = jnp.sum(jnp.where(sel1, p, 0.0), axis=-1, keepdims=True)

    mask = (sel0 | sel1).astype(jnp.bfloat16)
    ti = jax.lax.broadcasted_iota(jnp.int32, (T, T), 0)
    tj = jax.lax.broadcasted_iota(jnp.int32, (T, T), 1)
    tril = (tj < ti).astype(jnp.bfloat16)
    rank = jnp.dot(tril, mask, preferred_element_type=jnp.float32)

    counts = jnp.sum(mask.astype(jnp.float32), axis=0, keepdims=True)
    padded = jnp.ceil(counts * (1.0 / BM)) * BM
    p8 = jnp.broadcast_to(padded, (E, E)).astype(jnp.bfloat16)
    ei = jax.lax.broadcasted_iota(jnp.int32, (E, E), 0)
    ej = jax.lax.broadcasted_iota(jnp.int32, (E, E), 1)
    tru = (ei < ej).astype(jnp.bfloat16)
    base8 = jnp.dot(p8, tru, preferred_element_type=jnp.float32)
    base = base8[0:1, :]

    posv = base + rank
    d0 = jnp.sum(jnp.where(sel0, posv, 0.0), axis=-1, keepdims=True)
    d1 = jnp.sum(jnp.where(sel1, posv, 0.0), axis=-1, keepdims=True)

    dest_ref[:, 0:1] = d0.astype(jnp.int32)
    dest_ref[:, 1:2] = d1.astype(jnp.int32)
    w_ref[:, 0:1] = w0
    w_ref[:, 1:2] = w1
    pc_ref[...] = padded.astype(jnp.int32)


def _routing(x, gate_w):
    return pl.pallas_call(
        _routing_body,
        out_shape=(
            jax.ShapeDtypeStruct((T, K), jnp.int32),
            jax.ShapeDtypeStruct((T, K), jnp.float32),
            jax.ShapeDtypeStruct((1, E), jnp.int32),
        ),
    )(x, gate_w)


NW = 32
DCH = 16
DNCH = K * T // NW // DCH


def _dispatch(x, dest3):
    mesh = plsc.VectorSubcoreMesh(core_axis_name="c", subcore_axis_name="s")

    @functools.partial(
        pl.kernel,
        out_type=jax.ShapeDtypeStruct((C, H), jnp.float32),
        mesh=mesh,
        scratch_types=[
            pltpu.VMEM((DNCH, DCH), jnp.int32),
            pltpu.VMEM((DCH, H), jnp.float32),
            pltpu.SemaphoreType.DMA,
        ],
    )
    def k(x_hbm, d_hbm, o_hbm, idx_v, row_v, sem):
        wid = jax.lax.axis_index("s") * 2 + jax.lax.axis_index("c")
        tok0 = jax.lax.rem(wid * (DNCH * DCH), T)
        pltpu.sync_copy(d_hbm.at[wid], idx_v)

        @pl.loop(0, DNCH)
        def _(o):
            pltpu.sync_copy(x_hbm.at[pl.ds(tok0 + o * DCH, DCH)], row_v)
            pltpu.async_copy(row_v, o_hbm.at[idx_v.at[o]], sem).wait()

    return k(x, dest3)


def _grouped_body(eot_ref, x_ref, wi0_ref, wi1_ref, wo_ref, y_ref):
    x = x_ref[...].astype(jnp.bfloat16)
    w0 = wi0_ref[0].astype(jnp.bfloat16)
    w1 = wi1_ref[0].astype(jnp.bfloat16)
    g = jnp.dot(x, w0, preferred_element_type=jnp.float32)
    u = jnp.dot(x, w1, preferred_element_type=jnp.float32)
    h = (jax.nn.gelu(g, approximate=True) * u).astype(jnp.bfloat16)
    wo_b = wo_ref[0].astype(jnp.bfloat16)
    y_ref[0] = jnp.dot(h, wo_b, preferred_element_type=jnp.float32)


def _grouped(x_sorted, eot, wi_0, wi_1, wo):
    grid_spec = pltpu.PrefetchScalarGridSpec(
        num_scalar_prefetch=1,
        grid=(NJ, NT),
        in_specs=[
            pl.BlockSpec((BM, H), lambda j, i, eot: (i, 0)),
            pl.BlockSpec((1, H, IH), lambda j, i, eot: (eot[i], 0, j)),
            pl.BlockSpec((1, H, IH), lambda j, i, eot: (eot[i], 0, j)),
            pl.BlockSpec((1, IH, H), lambda j, i, eot: (eot[i], j, 0)),
        ],
        out_specs=pl.BlockSpec((1, BM, H), lambda j, i, eot: (j, i, 0)),
    )
    return pl.pallas_call(
        _grouped_body,
        grid_spec=grid_spec,
        out_shape=jax.ShapeDtypeStruct((NJ, C, H), jnp.float32),
    )(eot, x_sorted, wi_0, wi_1, wo)


CCH = 16
CNCH = T // NW // CCH


def _combine(y2, didx, w0b, w1b):
    mesh = plsc.VectorSubcoreMesh(core_axis_name="c", subcore_axis_name="s")

    @functools.partial(
        pl.kernel,
        out_type=jax.ShapeDtypeStruct((T, H), jnp.float32),
        mesh=mesh,
        scratch_types=[
            pltpu.VMEM((4, CNCH, CCH), jnp.int32),
            pltpu.VMEM((CCH, 16), jnp.float32),
            pltpu.VMEM((CCH, 16), jnp.float32),
            pltpu.VMEM((CCH, H), jnp.float32),
            pltpu.VMEM((CCH, H), jnp.float32),
            pltpu.VMEM((CCH, H), jnp.float32),
            pltpu.VMEM((CCH, H), jnp.float32),
            pltpu.VMEM((CCH, H), jnp.float32),
            pltpu.SemaphoreType.DMA,
        ],
    )
    def k(y_hbm, d_hbm, w0_hbm, w1_hbm, o_hbm,
          idx_v, w0_v, w1_v, a0_v, a1_v, b0_v, b1_v, o_v, sem):
        wid = jax.lax.axis_index("s") * 2 + jax.lax.axis_index("c")
        t0 = wid * (CNCH * CCH)
        pltpu.sync_copy(d_hbm.at[wid], idx_v)

        @pl.loop(0, CNCH)
        def _(ch):
            tslc = pl.ds(t0 + ch * CCH, CCH)
            pltpu.sync_copy(w0_hbm.at[tslc], w0_v)
            pltpu.sync_copy(w1_hbm.at[tslc], w1_v)
            pltpu.async_copy(y_hbm.at[idx_v.at[0, ch]], a0_v, sem).wait()
            pltpu.async_copy(y_hbm.at[idx_v.at[1, ch]], a1_v, sem).wait()
            pltpu.async_copy(y_hbm.at[idx_v.at[2, ch]], b0_v, sem).wait()
            pltpu.async_copy(y_hbm.at[idx_v.at[3, ch]], b1_v, sem).wait()

            @pl.loop(0, CCH)
            def _(r):
                w0 = w0_v[r, pl.ds(0, 16)]
                w1 = w1_v[r, pl.ds(0, 16)]

                @pl.loop(0, H, step=16)
                def _(c):
                    cs = pl.ds(c, 16)
                    a = a0_v[r, cs] + a1_v[r, cs]
                    b = b0_v[r, cs] + b1_v[r, cs]
                    o_v[r, cs] = a * w0 + b * w1

            pltpu.sync_copy(o_v, o_hbm.at[tslc])

    return k(y2, didx, w0b, w1b)


def kernel(hidden_states, gate_w, wi_0, wi_1, wo):
    dest, wts, pc = _routing(hidden_states, gate_w)

    tiles_pe = pc[0] // BM
    cumt = jnp.cumsum(tiles_pe)
    eot = jnp.minimum(
        jnp.searchsorted(cumt, jnp.arange(NT, dtype=jnp.int32), side="right"),
        E - 1,
    ).astype(jnp.int32)

    dest3 = dest.T.reshape(NW, DNCH, DCH)
    x_sorted = _dispatch(hidden_states, dest3)

    y = _grouped(x_sorted, eot, wi_0, wi_1, wo)

    y2 = y.reshape(NJ * C, H)
    d0 = dest[:, 0]
    d1 = dest[:, 1]
    didx = jnp.stack([d0, d0 + C, d1, d1 + C], axis=0)
    didx = didx.reshape(4, NW, CNCH * CCH).transpose(1, 0, 2)
    didx = didx.reshape(NW, 4, CNCH, CCH)
    w0b = jnp.broadcast_to(wts[:, 0:1], (T, 16))
    w1b = jnp.broadcast_to(wts[:, 1:2], (T, 16))
    return _combine(y2, didx, w0b, w1b)


# speedup vs baseline: 1.7091x; 1.0878x over previous
import functools

import jax
import jax.numpy as jnp
from jax.experimental import pallas as pl
from jax.experimental.pallas import tpu as pltpu
from jax.experimental.pallas import tpu_sc as plsc

T, H, I, E, K = 2048, 1024, 4096, 8, 2
SOFTCAP = 30.0
BM = 256
C = T * K + E * BM
NT = C // BM
NJ = 2
IH = I // NJ


def _routing_body(x_ref, g_ref, dest_ref, w_ref, pc_ref):
    x = x_ref[...]
    gw = g_ref[...]
    logits = jnp.dot(x.astype(jnp.bfloat16), gw.astype(jnp.bfloat16),
                     preferred_element_type=jnp.float32)
    logits = jnp.tanh(logits * (1.0 / SOFTCAP)) * SOFTCAP

    m = jnp.max(logits, axis=-1, keepdims=True)
    ex = jnp.exp(logits - m)
    p = ex / jnp.sum(ex, axis=-1, keepdims=True)

    lane = jax.lax.broadcasted_iota(jnp.int32, (T, E), 1)
    big = jnp.int32(E)
    c0 = jnp.where(logits == m, lane, big)
    i0 = jnp.min(c0, axis=-1, keepdims=True)
    sel0 = lane == i0
    l1 = jnp.where(sel0, -jnp.inf, logits)
    m1 = jnp.max(l1, axis=-1, keepdims=True)
    c1 = jnp.where(l1 == m1, lane, big)
    i1 = jnp.min(c1, axis=-1, keepdims=True)
    sel1 = lane == i1

    w0 = jnp.sum(jnp.where(sel0, p, 0.0), axis=-1, keepdims=True)
    w1 = jnp.sum(jnp.where(sel1, p, 0.0), axis=-1, keepdims=True)

    mask = (sel0 | sel1).astype(jnp.bfloat16)
    ti = jax.lax.broadcasted_iota(jnp.int32, (T, T), 0)
    tj = jax.lax.broadcasted_iota(jnp.int32, (T, T), 1)
    tril = (tj < ti).astype(jnp.bfloat16)
    rank = jnp.dot(tril, mask, preferred_element_type=jnp.float32)

    counts = jnp.sum(mask.astype(jnp.float32), axis=0, keepdims=True)
    padded = jnp.ceil(counts * (1.0 / BM)) * BM
    p8 = jnp.broadcast_to(padded, (E, E)).astype(jnp.bfloat16)
    ei = jax.lax.broadcasted_iota(jnp.int32, (E, E), 0)
    ej = jax.lax.broadcasted_iota(jnp.int32, (E, E), 1)
    tru = (ei < ej).astype(jnp.bfloat16)
    base8 = jnp.dot(p8, tru, preferred_element_type=jnp.float32)
    base = base8[0:1, :]

    posv = base + rank
    d0 = jnp.sum(jnp.where(sel0, posv, 0.0), axis=-1, keepdims=True)
    d1 = jnp.sum(jnp.where(sel1, posv, 0.0), axis=-1, keepdims=True)

    dest_ref[:, 0:1] = d0.astype(jnp.int32)
    dest_ref[:, 1:2] = d1.astype(jnp.int32)
    w_ref[:, 0:1] = w0
    w_ref[:, 1:2] = w1
    pc_ref[...] = padded.astype(jnp.int32)


def _routing(x, gate_w):
    return pl.pallas_call(
        _routing_body,
        out_shape=(
            jax.ShapeDtypeStruct((T, K), jnp.int32),
            jax.ShapeDtypeStruct((T, K), jnp.float32),
            jax.ShapeDtypeStruct((1, E), jnp.int32),
        ),
    )(x, gate_w)


NW = 32
DCH = 16
DNCH = K * T // NW // DCH


def _dispatch(x, dest3):
    mesh = plsc.VectorSubcoreMesh(core_axis_name="c", subcore_axis_name="s")

    @functools.partial(
        pl.kernel,
        out_type=jax.ShapeDtypeStruct((C, H), jnp.float32),
        mesh=mesh,
        scratch_types=[
            pltpu.VMEM((DNCH, DCH), jnp.int32),
            pltpu.VMEM((DCH, H), jnp.float32),
            pltpu.SemaphoreType.DMA,
        ],
    )
    def k(x_hbm, d_hbm, o_hbm, idx_v, row_v, sem):
        wid = jax.lax.axis_index("s") * 2 + jax.lax.axis_index("c")
        tok0 = jax.lax.rem(wid * (DNCH * DCH), T)
        pltpu.sync_copy(d_hbm.at[wid], idx_v)

        @pl.loop(0, DNCH)
        def _(o):
            pltpu.sync_copy(x_hbm.at[pl.ds(tok0 + o * DCH, DCH)], row_v)
            pltpu.async_copy(row_v, o_hbm.at[idx_v.at[o]], sem).wait()

    return k(x, dest3)


def _grouped_body(meta_ref, x_ref, wi0_ref, wi1_ref, wo_ref, y_ref):
    i = pl.program_id(1)

    @pl.when(i < meta_ref[NT])
    def _():
        x = x_ref[...].astype(jnp.bfloat16)
        w0 = wi0_ref[0].astype(jnp.bfloat16)
        w1 = wi1_ref[0].astype(jnp.bfloat16)
        g = jnp.dot(x, w0, preferred_element_type=jnp.float32)
        u = jnp.dot(x, w1, preferred_element_type=jnp.float32)
        h = (jax.nn.gelu(g, approximate=True) * u).astype(jnp.bfloat16)
        wo_b = wo_ref[0].astype(jnp.bfloat16)
        y_ref[0] = jnp.dot(h, wo_b, preferred_element_type=jnp.float32)


def _grouped(x_sorted, meta, wi_0, wi_1, wo):
    grid_spec = pltpu.PrefetchScalarGridSpec(
        num_scalar_prefetch=1,
        grid=(NJ, NT),
        in_specs=[
            pl.BlockSpec((BM, H), lambda j, i, meta: (i, 0)),
            pl.BlockSpec((1, H, IH), lambda j, i, meta: (meta[i], 0, j)),
            pl.BlockSpec((1, H, IH), lambda j, i, meta: (meta[i], 0, j)),
            pl.BlockSpec((1, IH, H), lambda j, i, meta: (meta[i], j, 0)),
        ],
        out_specs=pl.BlockSpec((1, BM, H), lambda j, i, meta: (j, i, 0)),
    )
    return pl.pallas_call(
        _grouped_body,
        grid_spec=grid_spec,
        out_shape=jax.ShapeDtypeStruct((NJ, C, H), jnp.float32),
        compiler_params=pltpu.CompilerParams(
            vmem_limit_bytes=63 * 1024 * 1024,
        ),
    )(meta, x_sorted, wi_0, wi_1, wo)


CCH = 16
CNCH = T // NW // CCH


def _combine(y2, didx, w0b, w1b):
    mesh = plsc.VectorSubcoreMesh(core_axis_name="c", subcore_axis_name="s")

    @functools.partial(
        pl.kernel,
        out_type=jax.ShapeDtypeStruct((T, H), jnp.float32),
        mesh=mesh,
        scratch_types=[
            pltpu.VMEM((4, CNCH, CCH), jnp.int32),
            pltpu.VMEM((CCH, 16), jnp.float32),
            pltpu.VMEM((CCH, 16), jnp.float32),
            pltpu.VMEM((CCH, H), jnp.float32),
            pltpu.VMEM((CCH, H), jnp.float32),
            pltpu.VMEM((CCH, H), jnp.float32),
            pltpu.VMEM((CCH, H), jnp.float32),
            pltpu.VMEM((CCH, H), jnp.float32),
            pltpu.SemaphoreType.DMA,
        ],
    )
    def k(y_hbm, d_hbm, w0_hbm, w1_hbm, o_hbm,
          idx_v, w0_v, w1_v, a0_v, a1_v, b0_v, b1_v, o_v, sem):
        wid = jax.lax.axis_index("s") * 2 + jax.lax.axis_index("c")
        t0 = wid * (CNCH * CCH)
        pltpu.sync_copy(d_hbm.at[wid], idx_v)

        @pl.loop(0, CNCH)
        def _(ch):
            tslc = pl.ds(t0 + ch * CCH, CCH)
            pltpu.sync_copy(w0_hbm.at[tslc], w0_v)
            pltpu.sync_copy(w1_hbm.at[tslc], w1_v)
            pltpu.async_copy(y_hbm.at[idx_v.at[0, ch]], a0_v, sem).wait()
            pltpu.async_copy(y_hbm.at[idx_v.at[1, ch]], a1_v, sem).wait()
            pltpu.async_copy(y_hbm.at[idx_v.at[2, ch]], b0_v, sem).wait()
            pltpu.async_copy(y_hbm.at[idx_v.at[3, ch]], b1_v, sem).wait()

            @pl.loop(0, CCH)
            def _(r):
                w0 = w0_v[r, pl.ds(0, 16)]
                w1 = w1_v[r, pl.ds(0, 16)]

                @pl.loop(0, H, step=16)
                def _(c):
                    cs = pl.ds(c, 16)
                    a = a0_v[r, cs] + a1_v[r, cs]
                    b = b0_v[r, cs] + b1_v[r, cs]
                    o_v[r, cs] = a * w0 + b * w1

            pltpu.sync_copy(o_v, o_hbm.at[tslc])

    return k(y2, didx, w0b, w1b)


def kernel(hidden_states, gate_w, wi_0, wi_1, wo):
    dest, wts, pc = _routing(hidden_states, gate_w)

    tiles_pe = pc[0] // BM
    cumt = jnp.cumsum(tiles_pe)
    eot = jnp.minimum(
        jnp.searchsorted(cumt, jnp.arange(NT, dtype=jnp.int32), side="right"),
        E - 1,
    ).astype(jnp.int32)
    meta = jnp.concatenate([eot, cumt[-1:].astype(jnp.int32)])

    dest3 = dest.T.reshape(NW, DNCH, DCH)
    x_sorted = _dispatch(hidden_states, dest3)

    y = _grouped(x_sorted, meta, wi_0, wi_1, wo)

    y2 = y.reshape(NJ * C, H)
    d0 = dest[:, 0]
    d1 = dest[:, 1]
    didx = jnp.stack([d0, d0 + C, d1, d1 + C], axis=0)
    didx = didx.reshape(4, NW, CNCH * CCH).transpose(1, 0, 2)
    didx = didx.reshape(NW, 4, CNCH, CCH)
    w0b = jnp.broadcast_to(wts[:, 0:1], (T, 16))
    w1b = jnp.broadcast_to(wts[:, 1:2], (T, 16))
    return _combine(y2, didx, w0b, w1b)


# speedup vs baseline: 2.1218x; 1.2415x over previous
import functools

import jax
import jax.numpy as jnp
from jax.experimental import pallas as pl
from jax.experimental.pallas import tpu as pltpu
from jax.experimental.pallas import tpu_sc as plsc

T, H, I, E, K = 2048, 1024, 4096, 8, 2
SOFTCAP = 30.0
BM = 256
C = T * K + E * BM
NT = C // BM
NJ = 2
IH = I // NJ


def _routing_body(x_ref, g_ref, dest_ref, w_ref, pc_ref):
    x = x_ref[...]
    gw = g_ref[...]
    logits = jnp.dot(x.astype(jnp.bfloat16), gw.astype(jnp.bfloat16),
                     preferred_element_type=jnp.float32)
    logits = jnp.tanh(logits * (1.0 / SOFTCAP)) * SOFTCAP

    m = jnp.max(logits, axis=-1, keepdims=True)
    ex = jnp.exp(logits - m)
    p = ex / jnp.sum(ex, axis=-1, keepdims=True)

    lane = jax.lax.broadcasted_iota(jnp.int32, (T, E), 1)
    big = jnp.int32(E)
    c0 = jnp.where(logits == m, lane, big)
    i0 = jnp.min(c0, axis=-1, keepdims=True)
    sel0 = lane == i0
    l1 = jnp.where(sel0, -jnp.inf, logits)
    m1 = jnp.max(l1, axis=-1, keepdims=True)
    c1 = jnp.where(l1 == m1, lane, big)
    i1 = jnp.min(c1, axis=-1, keepdims=True)
    sel1 = lane == i1

    w0 = jnp.sum(jnp.where(sel0, p, 0.0), axis=-1, keepdims=True)
    w1 = jnp.sum(jnp.where(sel1, p, 0.0), axis=-1, keepdims=True)

    mask = (sel0 | sel1).astype(jnp.bfloat16)
    ti = jax.lax.broadcasted_iota(jnp.int32, (T, T), 0)
    tj = jax.lax.broadcasted_iota(jnp.int32, (T, T), 1)
    tril = (tj < ti).astype(jnp.bfloat16)
    rank = jnp.dot(tril, mask, preferred_element_type=jnp.float32)

    counts = jnp.sum(mask.astype(jnp.float32), axis=0, keepdims=True)
    padded = jnp.ceil(counts * (1.0 / BM)) * BM
    p8 = jnp.broadcast_to(padded, (E, E)).astype(jnp.bfloat16)
    ei = jax.lax.broadcasted_iota(jnp.int32, (E, E), 0)
    ej = jax.lax.broadcasted_iota(jnp.int32, (E, E), 1)
    tru = (ei < ej).astype(jnp.bfloat16)
    base8 = jnp.dot(p8, tru, preferred_element_type=jnp.float32)
    base = base8[0:1, :]

    posv = base + rank
    d0 = jnp.sum(jnp.where(sel0, posv, 0.0), axis=-1, keepdims=True)
    d1 = jnp.sum(jnp.where(sel1, posv, 0.0), axis=-1, keepdims=True)

    dest_ref[:, 0:1] = d0.astype(jnp.int32)
    dest_ref[:, 1:2] = d1.astype(jnp.int32)
    w_ref[:, 0:1] = w0
    w_ref[:, 1:2] = w1
    pc_ref[...] = padded.astype(jnp.int32)


def _routing(x, gate_w):
    return pl.pallas_call(
        _routing_body,
        out_shape=(
            jax.ShapeDtypeStruct((T, K), jnp.int32),
            jax.ShapeDtypeStruct((T, K), jnp.float32),
            jax.ShapeDtypeStruct((1, E), jnp.int32),
        ),
    )(x, gate_w)


NW = 32
DCH = 16
DNCH = K * T // NW // DCH


def _dispatch(x, dest3):
    mesh = plsc.VectorSubcoreMesh(core_axis_name="c", subcore_axis_name="s")

    @functools.partial(
        pl.kernel,
        out_type=jax.ShapeDtypeStruct((C, H), jnp.float32),
        mesh=mesh,
        scratch_types=[
            pltpu.VMEM((DNCH, DCH), jnp.int32),
            pltpu.VMEM((DCH, H), jnp.float32),
            pltpu.SemaphoreType.DMA,
        ],
    )
    def k(x_hbm, d_hbm, o_hbm, idx_v, row_v, sem):
        wid = jax.lax.axis_index("s") * 2 + jax.lax.axis_index("c")
        tok0 = jax.lax.rem(wid * (DNCH * DCH), T)
        pltpu.sync_copy(d_hbm.at[wid], idx_v)

        @pl.loop(0, DNCH)
        def _(o):
            pltpu.sync_copy(x_hbm.at[pl.ds(tok0 + o * DCH, DCH)], row_v)
            pltpu.async_copy(row_v, o_hbm.at[idx_v.at[o]], sem).wait()

    return k(x, dest3)


def _grouped_outer(meta_ref, x_hbm, wi0_hbm, wi1_hbm, wo_hbm, y_hbm):
    def body(x_ref, wi0_ref, wi1_ref, wo_ref, y_ref):
        x = x_ref[...].astype(jnp.bfloat16)
        w0 = wi0_ref[0].astype(jnp.bfloat16)
        w1 = wi1_ref[0].astype(jnp.bfloat16)
        g = jnp.dot(x, w0, preferred_element_type=jnp.float32)
        u = jnp.dot(x, w1, preferred_element_type=jnp.float32)
        h = (jax.nn.gelu(g, approximate=True) * u).astype(jnp.bfloat16)
        wo_b = wo_ref[0].astype(jnp.bfloat16)
        y_ref[0] = jnp.dot(h, wo_b, preferred_element_type=jnp.float32)

    wbuf = pl.Buffered(buffer_count=2, use_lookahead=True)
    pltpu.emit_pipeline(
        body,
        grid=(NJ, meta_ref[NT]),
        in_specs=[
            pl.BlockSpec((BM, H), lambda j, i: (i, 0)),
            pl.BlockSpec((1, H, IH), lambda j, i: (meta_ref[i], 0, j),
                         pipeline_mode=wbuf),
            pl.BlockSpec((1, H, IH), lambda j, i: (meta_ref[i], 0, j),
                         pipeline_mode=wbuf),
            pl.BlockSpec((1, IH, H), lambda j, i: (meta_ref[i], j, 0),
                         pipeline_mode=wbuf),
        ],
        out_specs=[pl.BlockSpec((1, BM, H), lambda j, i: (j, i, 0))],
    )(x_hbm, wi0_hbm, wi1_hbm, wo_hbm, y_hbm)


def _grouped(x_sorted, meta, wi_0, wi_1, wo):
    return pl.pallas_call(
        _grouped_outer,
        in_specs=[
            pl.BlockSpec(memory_space=pltpu.SMEM),
            pl.BlockSpec(memory_space=pltpu.HBM),
            pl.BlockSpec(memory_space=pltpu.HBM),
            pl.BlockSpec(memory_space=pltpu.HBM),
            pl.BlockSpec(memory_space=pltpu.HBM),
        ],
        out_specs=pl.BlockSpec(memory_space=pltpu.HBM),
        out_shape=jax.ShapeDtypeStruct((NJ, C, H), jnp.float32),
        compiler_params=pltpu.CompilerParams(
            vmem_limit_bytes=63 * 1024 * 1024,
        ),
    )(meta, x_sorted, wi_0, wi_1, wo)


CCH = 16
CNCH = T // NW // CCH


def _combine(y2, didx, w0b, w1b):
    mesh = plsc.VectorSubcoreMesh(core_axis_name="c", subcore_axis_name="s")

    @functools.partial(
        pl.kernel,
        out_type=jax.ShapeDtypeStruct((T, H), jnp.float32),
        mesh=mesh,
        scratch_types=[
            pltpu.VMEM((4, CNCH, CCH), jnp.int32),
            pltpu.VMEM((CCH, 16), jnp.float32),
            pltpu.VMEM((CCH, 16), jnp.float32),
            pltpu.VMEM((CCH, H), jnp.float32),
            pltpu.VMEM((CCH, H), jnp.float32),
            pltpu.VMEM((CCH, H), jnp.float32),
            pltpu.VMEM((CCH, H), jnp.float32),
            pltpu.VMEM((CCH, H), jnp.float32),
            pltpu.SemaphoreType.DMA,
        ],
    )
    def k(y_hbm, d_hbm, w0_hbm, w1_hbm, o_hbm,
          idx_v, w0_v, w1_v, a0_v, a1_v, b0_v, b1_v, o_v, sem):
        wid = jax.lax.axis_index("s") * 2 + jax.lax.axis_index("c")
        t0 = wid * (CNCH * CCH)
        pltpu.sync_copy(d_hbm.at[wid], idx_v)

        @pl.loop(0, CNCH)
        def _(ch):
            tslc = pl.ds(t0 + ch * CCH, CCH)
            pltpu.sync_copy(w0_hbm.at[tslc], w0_v)
            pltpu.sync_copy(w1_hbm.at[tslc], w1_v)
            pltpu.async_copy(y_hbm.at[idx_v.at[0, ch]], a0_v, sem).wait()
            pltpu.async_copy(y_hbm.at[idx_v.at[1, ch]], a1_v, sem).wait()
            pltpu.async_copy(y_hbm.at[idx_v.at[2, ch]], b0_v, sem).wait()
            pltpu.async_copy(y_hbm.at[idx_v.at[3, ch]], b1_v, sem).wait()

            @pl.loop(0, CCH)
            def _(r):
                w0 = w0_v[r, pl.ds(0, 16)]
                w1 = w1_v[r, pl.ds(0, 16)]

                @pl.loop(0, H, step=16)
                def _(c):
                    cs = pl.ds(c, 16)
                    a = a0_v[r, cs] + a1_v[r, cs]
                    b = b0_v[r, cs] + b1_v[r, cs]
                    o_v[r, cs] = a * w0 + b * w1

            pltpu.sync_copy(o_v, o_hbm.at[tslc])

    return k(y2, didx, w0b, w1b)


def kernel(hidden_states, gate_w, wi_0, wi_1, wo):
    dest, wts, pc = _routing(hidden_states, gate_w)

    tiles_pe = pc[0] // BM
    cumt = jnp.cumsum(tiles_pe)
    eot = jnp.minimum(
        jnp.searchsorted(cumt, jnp.arange(NT, dtype=jnp.int32), side="right"),
        E - 1,
    ).astype(jnp.int32)
    meta = jnp.concatenate([eot, cumt[-1:].astype(jnp.int32)])

    dest3 = dest.T.reshape(NW, DNCH, DCH)
    x_sorted = _dispatch(hidden_states, dest3)

    y = _grouped(x_sorted, meta, wi_0, wi_1, wo)

    y2 = y.reshape(NJ * C, H)
    d0 = dest[:, 0]
    d1 = dest[:, 1]
    didx = jnp.stack([d0, d0 + C, d1, d1 + C], axis=0)
    didx = didx.reshape(4, NW, CNCH * CCH).transpose(1, 0, 2)
    didx = didx.reshape(NW, 4, CNCH, CCH)
    w0b = jnp.broadcast_to(wts[:, 0:1], (T, 16))
    w1b = jnp.broadcast_to(wts[:, 1:2], (T, 16))
    return _combine(y2, didx, w0b, w1b)


# speedup vs baseline: 2.1651x; 1.0204x over previous
import functools

import jax
import jax.numpy as jnp
from jax.experimental import pallas as pl
from jax.experimental.pallas import tpu as pltpu
from jax.experimental.pallas import tpu_sc as plsc

T, H, I, E, K = 2048, 1024, 4096, 8, 2
SOFTCAP = 30.0
BM = 256
C = T * K + E * BM
NT = C // BM
NJ = 2
IH = I // NJ


def _routing_body(x_ref, g_ref, dest_ref, w_ref, pc_ref):
    x = x_ref[...]
    gw = g_ref[...]
    logits = jnp.dot(x.astype(jnp.bfloat16), gw.astype(jnp.bfloat16),
                     preferred_element_type=jnp.float32)
    logits = jnp.tanh(logits * (1.0 / SOFTCAP)) * SOFTCAP

    m = jnp.max(logits, axis=-1, keepdims=True)
    ex = jnp.exp(logits - m)
    p = ex / jnp.sum(ex, axis=-1, keepdims=True)

    lane = jax.lax.broadcasted_iota(jnp.int32, (T, E), 1)
    big = jnp.int32(E)
    c0 = jnp.where(logits == m, lane, big)
    i0 = jnp.min(c0, axis=-1, keepdims=True)
    sel0 = lane == i0
    l1 = jnp.where(sel0, -jnp.inf, logits)
    m1 = jnp.max(l1, axis=-1, keepdims=True)
    c1 = jnp.where(l1 == m1, lane, big)
    i1 = jnp.min(c1, axis=-1, keepdims=True)
    sel1 = lane == i1

    w0 = jnp.sum(jnp.where(sel0, p, 0.0), axis=-1, keepdims=True)
    w1 = jnp.sum(jnp.where(sel1, p, 0.0), axis=-1, keepdims=True)

    mask = (sel0 | sel1).astype(jnp.bfloat16)
    ti = jax.lax.broadcasted_iota(jnp.int32, (T, T), 0)
    tj = jax.lax.broadcasted_iota(jnp.int32, (T, T), 1)
    tril = (tj < ti).astype(jnp.bfloat16)
    rank = jnp.dot(tril, mask, preferred_element_type=jnp.float32)

    counts = jnp.sum(mask.astype(jnp.float32), axis=0, keepdims=True)
    padded = jnp.ceil(counts * (1.0 / BM)) * BM
    p8 = jnp.broadcast_to(padded, (E, E)).astype(jnp.bfloat16)
    ei = jax.lax.broadcasted_iota(jnp.int32, (E, E), 0)
    ej = jax.lax.broadcasted_iota(jnp.int32, (E, E), 1)
    tru = (ei < ej).astype(jnp.bfloat16)
    base8 = jnp.dot(p8, tru, preferred_element_type=jnp.float32)
    base = base8[0:1, :]

    posv = base + rank
    d0 = jnp.sum(jnp.where(sel0, posv, 0.0), axis=-1, keepdims=True)
    d1 = jnp.sum(jnp.where(sel1, posv, 0.0), axis=-1, keepdims=True)

    dest_ref[:, 0:1] = d0.astype(jnp.int32)
    dest_ref[:, 1:2] = d1.astype(jnp.int32)
    w_ref[:, 0:1] = w0
    w_ref[:, 1:2] = w1
    pc_ref[...] = padded.astype(jnp.int32)


def _routing(x, gate_w):
    return pl.pallas_call(
        _routing_body,
        out_shape=(
            jax.ShapeDtypeStruct((T, K), jnp.int32),
            jax.ShapeDtypeStruct((T, K), jnp.float32),
            jax.ShapeDtypeStruct((1, E), jnp.int32),
        ),
    )(x, gate_w)


NW = 32
DCH = 16
DNCH = K * T // NW // DCH


def _dispatch(x, dest3, wflat):
    mesh = plsc.VectorSubcoreMesh(core_axis_name="c", subcore_axis_name="s")

    @functools.partial(
        pl.kernel,
        out_type=(
            jax.ShapeDtypeStruct((C, H), jnp.float32),
            jax.ShapeDtypeStruct((C, 128), jnp.float32),
        ),
        mesh=mesh,
        scratch_types=[
            pltpu.VMEM((DNCH, DCH), jnp.int32),
            pltpu.VMEM((DCH, H), jnp.float32),
            pltpu.VMEM((DCH, 128), jnp.float32),
            pltpu.SemaphoreType.DMA,
        ],
    )
    def k(x_hbm, d_hbm, w_hbm, o_hbm, ws_hbm, idx_v, row_v, wrow_v, sem):
        wid = jax.lax.axis_index("s") * 2 + jax.lax.axis_index("c")
        tok0 = jax.lax.rem(wid * (DNCH * DCH), T)
        pltpu.sync_copy(d_hbm.at[wid], idx_v)

        @pl.loop(0, DNCH)
        def _(o):
            pltpu.sync_copy(x_hbm.at[pl.ds(tok0 + o * DCH, DCH)], row_v)
            pltpu.sync_copy(
                w_hbm.at[pl.ds(wid * (DNCH * DCH) + o * DCH, DCH)], wrow_v)
            c1 = pltpu.async_copy(row_v, o_hbm.at[idx_v.at[o]], sem)
            c2 = pltpu.async_copy(wrow_v, ws_hbm.at[idx_v.at[o]], sem)
            c1.wait()
            c2.wait()

    return k(x, dest3, wflat)


def _grouped_outer(meta_ref, x_hbm, ws_hbm, wi0_hbm, wi1_hbm, wo_hbm, y_hbm):
    def body(x_ref, ws_ref, wi0_ref, wi1_ref, wo_ref, y_ref):
        x = x_ref[...].astype(jnp.bfloat16)
        w0 = wi0_ref[0].astype(jnp.bfloat16)
        w1 = wi1_ref[0].astype(jnp.bfloat16)
        g = jnp.dot(x, w0, preferred_element_type=jnp.float32)
        u = jnp.dot(x, w1, preferred_element_type=jnp.float32)
        h = (jax.nn.gelu(g, approximate=True) * u).astype(jnp.bfloat16)
        wo_b = wo_ref[0].astype(jnp.bfloat16)
        acc = jnp.dot(h, wo_b, preferred_element_type=jnp.float32)
        y_ref[0] = acc * ws_ref[:, 0:1]

    wbuf = pl.Buffered(buffer_count=2, use_lookahead=True)
    pltpu.emit_pipeline(
        body,
        grid=(NJ, meta_ref[NT]),
        in_specs=[
            pl.BlockSpec((BM, H), lambda j, i: (i, 0)),
            pl.BlockSpec((BM, 128), lambda j, i: (i, 0)),
            pl.BlockSpec((1, H, IH), lambda j, i: (meta_ref[i], 0, j),
                         pipeline_mode=wbuf),
            pl.BlockSpec((1, H, IH), lambda j, i: (meta_ref[i], 0, j),
                         pipeline_mode=wbuf),
            pl.BlockSpec((1, IH, H), lambda j, i: (meta_ref[i], j, 0),
                         pipeline_mode=wbuf),
        ],
        out_specs=[pl.BlockSpec((1, BM, H), lambda j, i: (j, i, 0))],
    )(x_hbm, ws_hbm, wi0_hbm, wi1_hbm, wo_hbm, y_hbm)


def _grouped(x_sorted, w_sorted, meta, wi_0, wi_1, wo):
    return pl.pallas_call(
        _grouped_outer,
        in_specs=[
            pl.BlockSpec(memory_space=pltpu.SMEM),
            pl.BlockSpec(memory_space=pltpu.HBM),
            pl.BlockSpec(memory_space=pltpu.HBM),
            pl.BlockSpec(memory_space=pltpu.HBM),
            pl.BlockSpec(memory_space=pltpu.HBM),
            pl.BlockSpec(memory_space=pltpu.HBM),
        ],
        out_specs=pl.BlockSpec(memory_space=pltpu.HBM),
        out_shape=jax.ShapeDtypeStruct((NJ, C, H), jnp.float32),
        compiler_params=pltpu.CompilerParams(
            vmem_limit_bytes=63 * 1024 * 1024,
        ),
    )(meta, x_sorted, w_sorted, wi_0, wi_1, wo)


CCH = 16
CNCH = T // NW // CCH


def _combine(y2, didx):
    mesh = plsc.VectorSubcoreMesh(core_axis_name="c", subcore_axis_name="s")

    @functools.partial(
        pl.kernel,
        out_type=jax.ShapeDtypeStruct((T, H), jnp.float32),
        mesh=mesh,
        scratch_types=[
            pltpu.VMEM((4, CNCH, CCH), jnp.int32),
            pltpu.VMEM((CCH, H), jnp.float32),
            pltpu.VMEM((CCH, H), jnp.float32),
            pltpu.VMEM((CCH, H), jnp.float32),
            pltpu.VMEM((CCH, H), jnp.float32),
            pltpu.VMEM((CCH, H), jnp.float32),
            pltpu.SemaphoreType.DMA,
        ],
    )
    def k(y_hbm, d_hbm, o_hbm, idx_v, a0_v, a1_v, b0_v, b1_v, o_v, sem):
        wid = jax.lax.axis_index("s") * 2 + jax.lax.axis_index("c")
        t0 = wid * (CNCH * CCH)
        pltpu.sync_copy(d_hbm.at[wid], idx_v)

        @pl.loop(0, CNCH)
        def _(ch):
            tslc = pl.ds(t0 + ch * CCH, CCH)
            c1 = pltpu.async_copy(y_hbm.at[idx_v.at[0, ch]], a0_v, sem)
            c2 = pltpu.async_copy(y_hbm.at[idx_v.at[1, ch]], a1_v, sem)
            c3 = pltpu.async_copy(y_hbm.at[idx_v.at[2, ch]], b0_v, sem)
            c4 = pltpu.async_copy(y_hbm.at[idx_v.at[3, ch]], b1_v, sem)
            c1.wait()
            c2.wait()
            c3.wait()
            c4.wait()

            @pl.loop(0, CCH)
            def _(r):
                @pl.loop(0, H, step=16)
                def _(c):
                    cs = pl.ds(c, 16)
                    o_v[r, cs] = ((a0_v[r, cs] + a1_v[r, cs])
                                  + (b0_v[r, cs] + b1_v[r, cs]))

            pltpu.sync_copy(o_v, o_hbm.at[tslc])

    return k(y2, didx)


def kernel(hidden_states, gate_w, wi_0, wi_1, wo):
    dest, wts, pc = _routing(hidden_states, gate_w)

    tiles_pe = pc[0] // BM
    cumt = jnp.cumsum(tiles_pe)
    eot = jnp.minimum(
        jnp.searchsorted(cumt, jnp.arange(NT, dtype=jnp.int32), side="right"),
        E - 1,
    ).astype(jnp.int32)
    meta = jnp.concatenate([eot, cumt[-1:].astype(jnp.int32)])

    dest3 = dest.T.reshape(NW, DNCH, DCH)
    wflat = jnp.broadcast_to(wts.T.reshape(K * T, 1), (K * T, 128))
    x_sorted, w_sorted = _dispatch(hidden_states, dest3, wflat)

    y = _grouped(x_sorted, w_sorted, meta, wi_0, wi_1, wo)

    y2 = y.reshape(NJ * C, H)
    d0 = dest[:, 0]
    d1 = dest[:, 1]
    didx = jnp.stack([d0, d0 + C, d1, d1 + C], axis=0)
    didx = didx.reshape(4, NW, CNCH * CCH).transpose(1, 0, 2)
    didx = didx.reshape(NW, 4, CNCH, CCH)
    return _combine(y2, didx)
